# baseline (speedup 1.0000x reference)
import functools

import jax
import jax.numpy as jnp
from jax import lax
from jax.experimental import pallas as pl
from jax.experimental.pallas import tpu as pltpu
from jax.experimental.pallas import tpu_sc as plsc

D_MODEL = 64
NUM_CORES = 2
NUM_SUBCORES = 16
NUM_WORKERS = NUM_CORES * NUM_SUBCORES
CHUNK = 256
NBUF = 2
LANES = 16


@functools.lru_cache(maxsize=None)
def _make_lookup(B: int, H: int):
    total = B * H
    assert total % (NUM_WORKERS * CHUNK * NBUF) == 0 and B % CHUNK == 0
    per_w = total // NUM_WORKERS
    n_chunks = per_w // CHUNK
    n_rounds = n_chunks // NBUF
    mesh = plsc.VectorSubcoreMesh(
        core_axis_name="c", subcore_axis_name="s",
        num_cores=NUM_CORES, num_subcores=NUM_SUBCORES)

    @functools.partial(
        pl.kernel,
        out_type=jax.ShapeDtypeStruct((H, D_MODEL, B), jnp.float32),
        mesh=mesh,
        scratch_types=[
            pltpu.VMEM((per_w,), jnp.int32),
            pltpu.VMEM((NBUF, CHUNK, D_MODEL), jnp.float32),
            pltpu.VMEM((NBUF, D_MODEL, CHUNK), jnp.float32),
        ] + [pltpu.SemaphoreType.DMA] * (2 * NBUF),
        compiler_params=pltpu.CompilerParams(
            use_tc_tiling_on_sc=False, needs_layout_passes=False),
    )
    def lookup(table_hbm, idx_hbm, out_hbm, idx_v, rows_v, tbuf_v, *sems):
        sem_g = sems[:NBUF]
        sem_o = sems[NBUF:]
        wid = lax.axis_index("s") * NUM_CORES + lax.axis_index("c")
        base = wid * per_w
        pltpu.sync_copy(idx_hbm.at[pl.ds(base, per_w)], idx_v)

        def gather_desc(i, b):
            return pltpu.make_async_copy(
                table_hbm.at[idx_v.at[pl.ds(i * CHUNK, CHUNK)]],
                rows_v.at[b], sem_g[b])

        def out_desc(i, b):
            i0 = base + i * CHUNK
            h = i0 // B
            b0 = i0 % B
            return pltpu.make_async_copy(
                tbuf_v.at[b], out_hbm.at[h, :, pl.ds(b0, CHUNK)], sem_o[b])

        def transpose(b):
            lane = lax.iota(jnp.int32, LANES)

            def jc_body(jc, carry):
                row_idx = jc * LANES + lane
                for d in range(D_MODEL):
                    col_idx = jnp.full((LANES,), d, jnp.int32)
                    vec = plsc.load_gather(rows_v.at[b], [row_idx, col_idx])
                    tbuf_v[b, d, pl.ds(jc * LANES, LANES)] = vec
                return carry

            lax.fori_loop(0, CHUNK // LANES, jc_body, 0)

        for b in range(NBUF):
            gather_desc(b, b).start()

        def body(r, carry):
            for b in range(NBUF):
                i = r * NBUF + b
                gather_desc(i, b).wait()

                @pl.when(r >= 1)
                def _():
                    out_desc(i, b).wait()

                transpose(b)

                @pl.when(r < n_rounds - 1)
                def _():
                    gather_desc(i + NBUF, b).start()

                out_desc(i, b).start()
            return carry

        lax.fori_loop(0, n_rounds, body, 0)

        for b in range(NBUF):
            out_desc((n_rounds - 1) * NBUF + b, b).wait()

    return lookup


@jax.jit
def kernel(x, weight):
    B, H = x.shape
    flat = jnp.transpose(x).reshape(B * H).astype(jnp.int32)
    out_t = _make_lookup(B, H)(weight, flat)
    return jnp.transpose(out_t, (2, 0, 1))


# speedup vs baseline: 1.6916x; 1.6916x over previous
import functools

import jax
import jax.numpy as jnp
from jax import lax
from jax.experimental import pallas as pl
from jax.experimental.pallas import tpu as pltpu
from jax.experimental.pallas import tpu_sc as plsc

D_MODEL = 64
NUM_CORES = 2
NUM_SUBCORES = 16
NUM_WORKERS = NUM_CORES * NUM_SUBCORES
CHUNK = 256
NBUF = 4


@functools.lru_cache(maxsize=None)
def _make_lookup(B: int):
    assert B % (NUM_WORKERS * CHUNK * NBUF) == 0
    b_per_w = B // NUM_WORKERS
    n_chunks = b_per_w // CHUNK
    n_rounds = n_chunks // NBUF
    mesh = plsc.VectorSubcoreMesh(
        core_axis_name="c", subcore_axis_name="s",
        num_cores=NUM_CORES, num_subcores=NUM_SUBCORES)

    @functools.partial(
        pl.kernel,
        out_type=jax.ShapeDtypeStruct((B, D_MODEL), jnp.float32),
        mesh=mesh,
        scratch_types=[
            pltpu.VMEM((b_per_w,), jnp.int32),
            pltpu.VMEM((NBUF, CHUNK, D_MODEL), jnp.float32),
        ] + [pltpu.SemaphoreType.DMA] * (2 * NBUF),
        compiler_params=pltpu.CompilerParams(use_tc_tiling_on_sc=False),
    )
    def lookup(table_hbm, idx_hbm, out_hbm, idx_v, rows_v, *sems):
        sem_g = sems[:NBUF]
        sem_o = sems[NBUF:]
        wid = lax.axis_index("s") * NUM_CORES + lax.axis_index("c")
        base = wid * b_per_w
        pltpu.sync_copy(idx_hbm.at[pl.ds(base, b_per_w)], idx_v)

        def gather_desc(i, b):
            return pltpu.make_async_copy(
                table_hbm.at[idx_v.at[pl.ds(i * CHUNK, CHUNK)]],
                rows_v.at[b], sem_g[b])

        def out_desc(i, b):
            return pltpu.make_async_copy(
                rows_v.at[b], out_hbm.at[pl.ds(base + i * CHUNK, CHUNK)],
                sem_o[b])

        for b in range(NBUF):
            gather_desc(b, b).start()

        def body(r, carry):
            for b in range(NBUF):
                i = r * NBUF + b
                gather_desc(i, b).wait()
                out_desc(i, b).start()
            for b in range(NBUF):
                i = r * NBUF + b
                out_desc(i, b).wait()
                gather_desc(i + NBUF, b).start()
            return carry

        lax.fori_loop(0, n_rounds - 1, body, 0)

        r_last = n_rounds - 1
        for b in range(NBUF):
            i = r_last * NBUF + b
            gather_desc(i, b).wait()
            out_desc(i, b).start()
        for b in range(NBUF):
            out_desc(r_last * NBUF + b, b).wait()

    return lookup


@jax.jit
def kernel(x, weight):
    B, H = x.shape
    flat = jnp.transpose(x).reshape(B * H).astype(jnp.int32)
    out_lin = _make_lookup(B * H)(weight, flat)
    return jnp.transpose(out_lin.reshape(H, B, D_MODEL), (1, 0, 2))
